# baseline (speedup 1.0000x reference)
import functools

import jax
import jax.numpy as jnp
from jax import lax
from jax.experimental import pallas as pl
from jax.experimental.pallas import tpu as pltpu
from jax.experimental.pallas import tpu_sc as plsc

N = 10000
NPAD = 10240
E = 320000
D = 128
DOUT = 17
DOUT_PAD = 32

NC = 2
NS = 16
NW = NC * NS
CH = 128
TOTCH = 2560
C0 = 152
C1 = 8
SCH = 40
SEC0 = (40, 40, 40, 32)
SEC1 = (8,)
NBUF = 2
GRP = SCH // NBUF
EPAD = TOTCH * CH
DEGCH = TOTCH // NW
RPT = NPAD // NS
PADNODE = N + 100

def _mesh():
    return plsc.VectorSubcoreMesh(core_axis_name="c", subcore_axis_name="s")


def _seg_sum(p, src_r, dst_r, d):
    @functools.partial(
        pl.kernel,
        mesh=_mesh(),
        out_type=jax.ShapeDtypeStruct((NC, NPAD, d), jnp.float32),
        compiler_params=pltpu.CompilerParams(use_tc_tiling_on_sc=(d == D)),
        scratch_types=[
            pltpu.VMEM((SCH, CH), jnp.int32),
            pltpu.VMEM((SCH, CH), jnp.int32),
            pltpu.VMEM((CH, d), jnp.float32),
            pltpu.VMEM((CH, d), jnp.float32),
            pltpu.VMEM_SHARED((NPAD, d), jnp.float32),
            pltpu.SemaphoreType.DMA,
            pltpu.SemaphoreType.DMA,
            pltpu.SemaphoreType.DMA,
            pltpu.SemaphoreType.DMA,
        ],
    )
    def seg_kernel(p_hbm, src_hbm, dst_hbm, out_hbm,
                   src_v, dst_v, r0, r1, agg_sh, g0, g1, s0, s1):
        cid = lax.axis_index("c")
        sid = lax.axis_index("s")
        bufs = (r0, r1)
        gsem = (g0, g1)
        ssem = (s0, s1)

        zv = jnp.zeros((16,), jnp.float32)
        for i in range(16):
            for jj in range(d // 16):
                r0[i, pl.ds(jj * 16, 16)] = zv

        def zbody(r, carry):
            pltpu.sync_copy(r0.at[pl.ds(0, 16)],
                            agg_sh.at[pl.ds(sid * RPT + r * 16, 16)])
            return carry
        lax.fori_loop(0, RPT // 16, zbody, 0)
        plsc.subcore_barrier()

        def pipeline(chunk0, sizes):
            base = chunk0
            for size in sizes:
                grp = size // NBUF
                pltpu.sync_copy(src_hbm.at[pl.ds(base, size)],
                                src_v.at[pl.ds(0, size)])
                pltpu.sync_copy(dst_hbm.at[pl.ds(base, size)],
                                dst_v.at[pl.ds(0, size)])
                base = base + size

                for b in range(NBUF):
                    pltpu.async_copy(p_hbm.at[src_v.at[b]], bufs[b], gsem[b])

                def body(g, carry):
                    for b in range(NBUF):
                        j = g * NBUF + b
                        pltpu.make_async_copy(p_hbm.at[src_v.at[j]],
                                              bufs[b], gsem[b]).wait()
                        pltpu.async_copy(bufs[b], agg_sh.at[dst_v.at[j]],
                                         ssem[b], add=True)
                    for b in range(NBUF):
                        j = g * NBUF + b
                        pltpu.make_async_copy(bufs[b], agg_sh.at[dst_v.at[j]],
                                              ssem[b]).wait()
                        pltpu.async_copy(p_hbm.at[src_v.at[j + NBUF]],
                                         bufs[b], gsem[b])
                    return carry
                lax.fori_loop(0, grp - 1, body, 0)

                for b in range(NBUF):
                    j = (grp - 1) * NBUF + b
                    pltpu.make_async_copy(p_hbm.at[src_v.at[j]],
                                          bufs[b], gsem[b]).wait()
                    pltpu.async_copy(bufs[b], agg_sh.at[dst_v.at[j]],
                                     ssem[b], add=True)
                for b in range(NBUF):
                    j = (grp - 1) * NBUF + b
                    pltpu.make_async_copy(bufs[b], agg_sh.at[dst_v.at[j]],
                                          ssem[b]).wait()

        @pl.when(cid == 0)
        def _():
            pipeline(sid * C0, SEC0)

        @pl.when(cid != 0)
        def _():
            pipeline(NS * C0 + sid * C1, SEC1)

        plsc.subcore_barrier()
        pltpu.sync_copy(agg_sh.at[pl.ds(sid * RPT, RPT)],
                        out_hbm.at[cid].at[pl.ds(sid * RPT, RPT)])

    return seg_kernel(p, src_r, dst_r)


def _degrees(dst_r):
    DW = DOUT_PAD
    @functools.partial(
        pl.kernel,
        mesh=_mesh(),
        out_type=jax.ShapeDtypeStruct((NC, NPAD, DW), jnp.float32),
        compiler_params=pltpu.CompilerParams(use_tc_tiling_on_sc=False),
        scratch_types=[
            pltpu.VMEM((DEGCH, CH), jnp.int32),
            pltpu.VMEM((CH, DW), jnp.float32),
            pltpu.VMEM((16, DW), jnp.float32),
            pltpu.VMEM_SHARED((NPAD, DW), jnp.float32),
        ],
    )
    def deg_kernel(dst_hbm, out_hbm, dst_v, ones_v, zero_v, deg_sh):
        cid = lax.axis_index("c")
        sid = lax.axis_index("s")
        wid = sid * NC + cid

        zv = jnp.zeros((16,), jnp.float32)
        ov = jnp.ones((16,), jnp.float32)
        for i in range(16):
            for jj in range(DW // 16):
                zero_v[i, pl.ds(jj * 16, 16)] = zv
        for i in range(CH):
            for jj in range(DW // 16):
                ones_v[i, pl.ds(jj * 16, 16)] = ov

        def zbody(r, carry):
            pltpu.sync_copy(zero_v, deg_sh.at[pl.ds(sid * RPT + r * 16, 16)])
            return carry
        lax.fori_loop(0, RPT // 16, zbody, 0)
        plsc.subcore_barrier()

        pltpu.sync_copy(dst_hbm.at[pl.ds(wid * DEGCH, DEGCH)], dst_v)

        def body(j, carry):
            pltpu.sync_copy(ones_v, deg_sh.at[dst_v.at[j]], add=True)
            return carry
        lax.fori_loop(0, DEGCH, body, 0)

        plsc.subcore_barrier()
        pltpu.sync_copy(deg_sh.at[pl.ds(sid * RPT, RPT)],
                        out_hbm.at[cid].at[pl.ds(sid * RPT, RPT)])

    return deg_kernel(dst_r)


_BM = 512


def _mm(h, w):
    K = h.shape[1]
    dout = w.shape[0]

    def body(h_ref, w_ref, o_ref):
        o_ref[...] = lax.dot_general(
            h_ref[...], w_ref[...], (((1,), (1,)), ((), ())),
            preferred_element_type=jnp.float32)

    return pl.pallas_call(
        body,
        grid=(NPAD // _BM,),
        in_specs=[pl.BlockSpec((_BM, K), lambda i: (i, 0)),
                  pl.BlockSpec((dout, K), lambda i: (0, 0))],
        out_specs=pl.BlockSpec((_BM, dout), lambda i: (i, 0)),
        out_shape=jax.ShapeDtypeStruct((NPAD, dout), jnp.float32),
    )(h, w)


def _combine(agg, deg, h, wr, bias, wl_next):
    d = agg.shape[2]
    dnext = wl_next.shape[0]

    dd = deg.shape[2]

    def body(agg_ref, deg_ref, h_ref, wr_ref, b_ref, wl_ref, hn_ref, pn_ref):
        dsum = deg_ref[0, :, 0:1] + deg_ref[1, :, 0:1]
        recip = 1.0 / jnp.maximum(dsum, 1.0)
        mean = (agg_ref[0] + agg_ref[1]) * recip
        z = mean + lax.dot_general(
            h_ref[...], wr_ref[...], (((1,), (1,)), ((), ())),
            preferred_element_type=jnp.float32) + b_ref[...]
        hn = jnp.maximum(z, 0.0)
        hn_ref[...] = hn
        pn_ref[...] = lax.dot_general(
            hn, wl_ref[...], (((1,), (1,)), ((), ())),
            preferred_element_type=jnp.float32)

    return pl.pallas_call(
        body,
        grid=(NPAD // _BM,),
        in_specs=[
            pl.BlockSpec((NC, _BM, d), lambda i: (0, i, 0)),
            pl.BlockSpec((NC, _BM, dd), lambda i: (0, i, 0)),
            pl.BlockSpec((_BM, D), lambda i: (i, 0)),
            pl.BlockSpec((d, D), lambda i: (0, 0)),
            pl.BlockSpec((1, d), lambda i: (0, 0)),
            pl.BlockSpec((dnext, d), lambda i: (0, 0)),
        ],
        out_specs=[pl.BlockSpec((_BM, d), lambda i: (i, 0)),
                   pl.BlockSpec((_BM, dnext), lambda i: (i, 0))],
        out_shape=[jax.ShapeDtypeStruct((NPAD, d), jnp.float32),
                   jax.ShapeDtypeStruct((NPAD, dnext), jnp.float32)],
    )(agg, deg, h, wr, bias, wl_next)


def _combine_last(agg, deg, h, wr_pad, bias_pad):
    d = agg.shape[2]

    dd = deg.shape[2]

    def body(agg_ref, deg_ref, h_ref, wr_ref, b_ref, o_ref):
        dsum = deg_ref[0, :, 0:1] + deg_ref[1, :, 0:1]
        recip = 1.0 / jnp.maximum(dsum, 1.0)
        mean = (agg_ref[0] + agg_ref[1]) * recip
        z = mean + lax.dot_general(
            h_ref[...], wr_ref[...], (((1,), (1,)), ((), ())),
            preferred_element_type=jnp.float32) + b_ref[...]
        o_ref[...] = jnp.maximum(z, 0.0)

    return pl.pallas_call(
        body,
        grid=(NPAD // _BM,),
        in_specs=[
            pl.BlockSpec((NC, _BM, d), lambda i: (0, i, 0)),
            pl.BlockSpec((NC, _BM, dd), lambda i: (0, i, 0)),
            pl.BlockSpec((_BM, D), lambda i: (i, 0)),
            pl.BlockSpec((d, D), lambda i: (0, 0)),
            pl.BlockSpec((1, d), lambda i: (0, 0)),
        ],
        out_specs=pl.BlockSpec((_BM, d), lambda i: (i, 0)),
        out_shape=jax.ShapeDtypeStruct((NPAD, d), jnp.float32),
    )(agg, deg, h, wr_pad, bias_pad)


def kernel(x, edge_index, Wl1, Wr1, b1, Wl2, Wr2, b2, Wl3, Wr3, b3,
           Wl4, Wr4, b4, Wl5, Wr5, b5):
    x_pad = jnp.pad(x, ((0, NPAD - N), (0, 0)))
    src_r = jnp.pad(edge_index[0], (0, EPAD - E),
                    constant_values=PADNODE).reshape(TOTCH, CH)
    dst_r = jnp.pad(edge_index[1], (0, EPAD - E),
                    constant_values=PADNODE).reshape(TOTCH, CH)

    Wl5p = jnp.pad(Wl5, ((0, DOUT_PAD - DOUT), (0, 0)))
    Wr5p = jnp.pad(Wr5, ((0, DOUT_PAD - DOUT), (0, 0)))
    b5p = jnp.pad(b5, (0, DOUT_PAD - DOUT)).reshape(1, DOUT_PAD)

    deg = _degrees(dst_r)

    h = x_pad
    p = _mm(x_pad, Wl1)
    agg = _seg_sum(p, src_r, dst_r, D)
    h, p = _combine(agg, deg, h, Wr1, b1.reshape(1, D), Wl2)
    agg = _seg_sum(p, src_r, dst_r, D)
    h, p = _combine(agg, deg, h, Wr2, b2.reshape(1, D), Wl3)
    agg = _seg_sum(p, src_r, dst_r, D)
    h, p = _combine(agg, deg, h, Wr3, b3.reshape(1, D), Wl4)
    agg = _seg_sum(p, src_r, dst_r, D)
    h, p = _combine(agg, deg, h, Wr4, b4.reshape(1, D), Wl5p)
    agg = _seg_sum(p, src_r, dst_r, DOUT_PAD)
    out = _combine_last(agg, deg, h, Wr5p, b5p)
    return out[:N, :DOUT]


# speedup vs baseline: 1.0013x; 1.0013x over previous
import functools

import jax
import jax.numpy as jnp
from jax import lax
from jax.experimental import pallas as pl
from jax.experimental.pallas import tpu as pltpu
from jax.experimental.pallas import tpu_sc as plsc

N = 10000
NPAD = 10240
E = 320000
D = 128
DOUT = 17
DOUT_PAD = 32

NC = 2
NS = 16
NW = NC * NS
CH = 128
TOTCH = 2560
C0 = 144
C1 = 16
SCH = 40
SEC0 = (40, 40, 40, 24)
SEC1 = (16,)
NBUF = 2
GRP = SCH // NBUF
EPAD = TOTCH * CH
DEGCH = TOTCH // NW
RPT = NPAD // NS
PADNODE = N + 100

def _mesh():
    return plsc.VectorSubcoreMesh(core_axis_name="c", subcore_axis_name="s")


def _seg_sum(p, src_r, dst_r, d):
    @functools.partial(
        pl.kernel,
        mesh=_mesh(),
        out_type=jax.ShapeDtypeStruct((NC, NPAD, d), jnp.float32),
        compiler_params=pltpu.CompilerParams(use_tc_tiling_on_sc=(d == D)),
        scratch_types=[
            pltpu.VMEM((SCH, CH), jnp.int32),
            pltpu.VMEM((SCH, CH), jnp.int32),
            pltpu.VMEM((CH, d), jnp.float32),
            pltpu.VMEM((CH, d), jnp.float32),
            pltpu.VMEM_SHARED((NPAD, d), jnp.float32),
            pltpu.SemaphoreType.DMA,
            pltpu.SemaphoreType.DMA,
            pltpu.SemaphoreType.DMA,
            pltpu.SemaphoreType.DMA,
        ],
    )
    def seg_kernel(p_hbm, src_hbm, dst_hbm, out_hbm,
                   src_v, dst_v, r0, r1, agg_sh, g0, g1, s0, s1):
        cid = lax.axis_index("c")
        sid = lax.axis_index("s")
        bufs = (r0, r1)
        gsem = (g0, g1)
        ssem = (s0, s1)

        zv = jnp.zeros((16,), jnp.float32)
        for i in range(16):
            for jj in range(d // 16):
                r0[i, pl.ds(jj * 16, 16)] = zv

        def zbody(r, carry):
            pltpu.sync_copy(r0.at[pl.ds(0, 16)],
                            agg_sh.at[pl.ds(sid * RPT + r * 16, 16)])
            return carry
        lax.fori_loop(0, RPT // 16, zbody, 0)
        plsc.subcore_barrier()

        def pipeline(chunk0, sizes):
            base = chunk0
            for size in sizes:
                grp = size // NBUF
                pltpu.sync_copy(src_hbm.at[pl.ds(base, size)],
                                src_v.at[pl.ds(0, size)])
                pltpu.sync_copy(dst_hbm.at[pl.ds(base, size)],
                                dst_v.at[pl.ds(0, size)])
                base = base + size

                for b in range(NBUF):
                    pltpu.async_copy(p_hbm.at[src_v.at[b]], bufs[b], gsem[b])

                def body(g, carry):
                    for b in range(NBUF):
                        j = g * NBUF + b
                        pltpu.make_async_copy(p_hbm.at[src_v.at[j]],
                                              bufs[b], gsem[b]).wait()
                        pltpu.async_copy(bufs[b], agg_sh.at[dst_v.at[j]],
                                         ssem[b], add=True)
                    for b in range(NBUF):
                        j = g * NBUF + b
                        pltpu.make_async_copy(bufs[b], agg_sh.at[dst_v.at[j]],
                                              ssem[b]).wait()
                        pltpu.async_copy(p_hbm.at[src_v.at[j + NBUF]],
                                         bufs[b], gsem[b])
                    return carry
                lax.fori_loop(0, grp - 1, body, 0)

                for b in range(NBUF):
                    j = (grp - 1) * NBUF + b
                    pltpu.make_async_copy(p_hbm.at[src_v.at[j]],
                                          bufs[b], gsem[b]).wait()
                    pltpu.async_copy(bufs[b], agg_sh.at[dst_v.at[j]],
                                     ssem[b], add=True)
                for b in range(NBUF):
                    j = (grp - 1) * NBUF + b
                    pltpu.make_async_copy(bufs[b], agg_sh.at[dst_v.at[j]],
                                          ssem[b]).wait()

        @pl.when(cid == 0)
        def _():
            pipeline(sid * C0, SEC0)

        @pl.when(cid != 0)
        def _():
            pipeline(NS * C0 + sid * C1, SEC1)

        plsc.subcore_barrier()
        pltpu.sync_copy(agg_sh.at[pl.ds(sid * RPT, RPT)],
                        out_hbm.at[cid].at[pl.ds(sid * RPT, RPT)])

    return seg_kernel(p, src_r, dst_r)


def _degrees(dst_r):
    DW = DOUT_PAD
    @functools.partial(
        pl.kernel,
        mesh=_mesh(),
        out_type=jax.ShapeDtypeStruct((NC, NPAD, DW), jnp.float32),
        compiler_params=pltpu.CompilerParams(use_tc_tiling_on_sc=False),
        scratch_types=[
            pltpu.VMEM((DEGCH, CH), jnp.int32),
            pltpu.VMEM((CH, DW), jnp.float32),
            pltpu.VMEM((16, DW), jnp.float32),
            pltpu.VMEM_SHARED((NPAD, DW), jnp.float32),
        ],
    )
    def deg_kernel(dst_hbm, out_hbm, dst_v, ones_v, zero_v, deg_sh):
        cid = lax.axis_index("c")
        sid = lax.axis_index("s")
        wid = sid * NC + cid

        zv = jnp.zeros((16,), jnp.float32)
        ov = jnp.ones((16,), jnp.float32)
        for i in range(16):
            for jj in range(DW // 16):
                zero_v[i, pl.ds(jj * 16, 16)] = zv
        for i in range(CH):
            for jj in range(DW // 16):
                ones_v[i, pl.ds(jj * 16, 16)] = ov

        def zbody(r, carry):
            pltpu.sync_copy(zero_v, deg_sh.at[pl.ds(sid * RPT + r * 16, 16)])
            return carry
        lax.fori_loop(0, RPT // 16, zbody, 0)
        plsc.subcore_barrier()

        pltpu.sync_copy(dst_hbm.at[pl.ds(wid * DEGCH, DEGCH)], dst_v)

        def body(j, carry):
            pltpu.sync_copy(ones_v, deg_sh.at[dst_v.at[j]], add=True)
            return carry
        lax.fori_loop(0, DEGCH, body, 0)

        plsc.subcore_barrier()
        pltpu.sync_copy(deg_sh.at[pl.ds(sid * RPT, RPT)],
                        out_hbm.at[cid].at[pl.ds(sid * RPT, RPT)])

    return deg_kernel(dst_r)


_BM = 512


def _mm(h, w):
    K = h.shape[1]
    dout = w.shape[0]

    def body(h_ref, w_ref, o_ref):
        o_ref[...] = lax.dot_general(
            h_ref[...], w_ref[...], (((1,), (1,)), ((), ())),
            preferred_element_type=jnp.float32)

    return pl.pallas_call(
        body,
        grid=(NPAD // _BM,),
        in_specs=[pl.BlockSpec((_BM, K), lambda i: (i, 0)),
                  pl.BlockSpec((dout, K), lambda i: (0, 0))],
        out_specs=pl.BlockSpec((_BM, dout), lambda i: (i, 0)),
        out_shape=jax.ShapeDtypeStruct((NPAD, dout), jnp.float32),
    )(h, w)


def _combine(agg, deg, h, wr, bias, wl_next):
    d = agg.shape[2]
    dnext = wl_next.shape[0]

    dd = deg.shape[2]

    def body(agg_ref, deg_ref, h_ref, wr_ref, b_ref, wl_ref, hn_ref, pn_ref):
        dsum = deg_ref[0, :, 0:1] + deg_ref[1, :, 0:1]
        recip = 1.0 / jnp.maximum(dsum, 1.0)
        mean = (agg_ref[0] + agg_ref[1]) * recip
        z = mean + lax.dot_general(
            h_ref[...], wr_ref[...], (((1,), (1,)), ((), ())),
            preferred_element_type=jnp.float32) + b_ref[...]
        hn = jnp.maximum(z, 0.0)
        hn_ref[...] = hn
        pn_ref[...] = lax.dot_general(
            hn, wl_ref[...], (((1,), (1,)), ((), ())),
            preferred_element_type=jnp.float32)

    return pl.pallas_call(
        body,
        grid=(NPAD // _BM,),
        in_specs=[
            pl.BlockSpec((NC, _BM, d), lambda i: (0, i, 0)),
            pl.BlockSpec((NC, _BM, dd), lambda i: (0, i, 0)),
            pl.BlockSpec((_BM, D), lambda i: (i, 0)),
            pl.BlockSpec((d, D), lambda i: (0, 0)),
            pl.BlockSpec((1, d), lambda i: (0, 0)),
            pl.BlockSpec((dnext, d), lambda i: (0, 0)),
        ],
        out_specs=[pl.BlockSpec((_BM, d), lambda i: (i, 0)),
                   pl.BlockSpec((_BM, dnext), lambda i: (i, 0))],
        out_shape=[jax.ShapeDtypeStruct((NPAD, d), jnp.float32),
                   jax.ShapeDtypeStruct((NPAD, dnext), jnp.float32)],
    )(agg, deg, h, wr, bias, wl_next)


def _combine_last(agg, deg, h, wr_pad, bias_pad):
    d = agg.shape[2]

    dd = deg.shape[2]

    def body(agg_ref, deg_ref, h_ref, wr_ref, b_ref, o_ref):
        dsum = deg_ref[0, :, 0:1] + deg_ref[1, :, 0:1]
        recip = 1.0 / jnp.maximum(dsum, 1.0)
        mean = (agg_ref[0] + agg_ref[1]) * recip
        z = mean + lax.dot_general(
            h_ref[...], wr_ref[...], (((1,), (1,)), ((), ())),
            preferred_element_type=jnp.float32) + b_ref[...]
        o_ref[...] = jnp.maximum(z, 0.0)

    return pl.pallas_call(
        body,
        grid=(NPAD // _BM,),
        in_specs=[
            pl.BlockSpec((NC, _BM, d), lambda i: (0, i, 0)),
            pl.BlockSpec((NC, _BM, dd), lambda i: (0, i, 0)),
            pl.BlockSpec((_BM, D), lambda i: (i, 0)),
            pl.BlockSpec((d, D), lambda i: (0, 0)),
            pl.BlockSpec((1, d), lambda i: (0, 0)),
        ],
        out_specs=pl.BlockSpec((_BM, d), lambda i: (i, 0)),
        out_shape=jax.ShapeDtypeStruct((NPAD, d), jnp.float32),
    )(agg, deg, h, wr_pad, bias_pad)


def kernel(x, edge_index, Wl1, Wr1, b1, Wl2, Wr2, b2, Wl3, Wr3, b3,
           Wl4, Wr4, b4, Wl5, Wr5, b5):
    x_pad = jnp.pad(x, ((0, NPAD - N), (0, 0)))
    src_r = jnp.pad(edge_index[0], (0, EPAD - E),
                    constant_values=PADNODE).reshape(TOTCH, CH)
    dst_r = jnp.pad(edge_index[1], (0, EPAD - E),
                    constant_values=PADNODE).reshape(TOTCH, CH)

    Wl5p = jnp.pad(Wl5, ((0, DOUT_PAD - DOUT), (0, 0)))
    Wr5p = jnp.pad(Wr5, ((0, DOUT_PAD - DOUT), (0, 0)))
    b5p = jnp.pad(b5, (0, DOUT_PAD - DOUT)).reshape(1, DOUT_PAD)

    deg = _degrees(dst_r)

    h = x_pad
    p = _mm(x_pad, Wl1)
    agg = _seg_sum(p, src_r, dst_r, D)
    h, p = _combine(agg, deg, h, Wr1, b1.reshape(1, D), Wl2)
    agg = _seg_sum(p, src_r, dst_r, D)
    h, p = _combine(agg, deg, h, Wr2, b2.reshape(1, D), Wl3)
    agg = _seg_sum(p, src_r, dst_r, D)
    h, p = _combine(agg, deg, h, Wr3, b3.reshape(1, D), Wl4)
    agg = _seg_sum(p, src_r, dst_r, D)
    h, p = _combine(agg, deg, h, Wr4, b4.reshape(1, D), Wl5p)
    agg = _seg_sum(p, src_r, dst_r, DOUT_PAD)
    out = _combine_last(agg, deg, h, Wr5p, b5p)
    return out[:N, :DOUT]
